# baseline (speedup 1.0000x reference)
import functools
import math

import jax
import jax.numpy as jnp
from jax import lax
from jax.experimental import pallas as pl
from jax.experimental.pallas import tpu as pltpu
from jax.experimental.pallas import tpu_sc as plsc

EB = 1024
NBV = 128
NW = 32
CH = 80
KR = 4


def _pack(m):
    c2 = m.shape[1] // 2
    mb = m.astype(jnp.bfloat16).reshape(m.shape[0], 2, c2)
    return pltpu.bitcast(mb, jnp.int32).reshape(m.shape[0], c2)


def _unpack(p):
    b = pltpu.bitcast(p.reshape(p.shape[0], 1, p.shape[1]), jnp.bfloat16)
    return b.reshape(p.shape[0], 2 * p.shape[1])


def _staircase_tables(dst_p, n_eb, nblk, s_max):
    lo = dst_p[0::EB] // NBV
    hi = dst_p[EB - 1::EB] // NBV
    nbs = jnp.arange(nblk, dtype=jnp.int32)
    ebf = jnp.searchsorted(hi, nbs, side="left").astype(jnp.int32)
    ebl = (jnp.searchsorted(lo, nbs, side="right") - 1).astype(jnp.int32)
    cnt = jnp.maximum(ebl - ebf + 1, 1)
    offs = jnp.concatenate([jnp.zeros((1,), jnp.int32),
                            jnp.cumsum(cnt)[:-1].astype(jnp.int32)])
    jj = jnp.arange(s_max, dtype=jnp.int32)
    nb_of = (jnp.searchsorted(offs, jj, side="right") - 1).astype(jnp.int32)
    within = jj - offs[nb_of]
    eb_of = jnp.clip(ebf[nb_of] + within, 0, n_eb - 1).astype(jnp.int32)
    first = (within == 0).astype(jnp.int32)
    last = (within == (cnt[nb_of] - 1)).astype(jnp.int32)
    return nb_of, eb_of, first, last


def _deg_body(n, nb_r, eb_r, fi_r, la_r, dst_r, out_r):
    s = pl.program_id(0)
    base = nb_r[s] * NBV
    dstv = dst_r[0]
    rows = lax.broadcasted_iota(jnp.int32, (NBV, EB), 0) + base
    t = (rows == jnp.broadcast_to(dstv, (NBV, EB))).astype(jnp.float32)
    c = jnp.sum(t, axis=1).reshape(1, 1, NBV)

    @pl.when(fi_r[s] == 1)
    def _():
        out_r[...] = c

    @pl.when(fi_r[s] == 0)
    def _():
        out_r[...] = out_r[...] + c

    @pl.when(la_r[s] == 1)
    def _():
        gidx = lax.broadcasted_iota(jnp.int32, (1, 1, NBV), 2) + base
        out_r[...] = jnp.where(gidx < n, lax.rsqrt(out_r[...] + 1.0), 0.0)


def _matmul_scale_body(x_r, w_r, d_r, out_r):
    m = jnp.dot(x_r[...].astype(jnp.bfloat16), w_r[...].astype(jnp.bfloat16),
                preferred_element_type=jnp.float32)
    out_r[...] = _pack(d_r[...] * m)


def _seg_body(nb_r, eb_r, fi_r, dst_r, ev_r, out_r):
    s = pl.program_id(0)
    base = nb_r[s] * NBV
    dstv = dst_r[0]
    rows = lax.broadcasted_iota(jnp.int32, (NBV, EB), 0) + base
    t = (rows == jnp.broadcast_to(dstv, (NBV, EB))).astype(jnp.bfloat16)
    evb = _unpack(ev_r[0])
    r = jnp.dot(t, evb, preferred_element_type=jnp.float32)[None]

    @pl.when(fi_r[s] == 1)
    def _():
        out_r[...] = r

    @pl.when(fi_r[s] == 0)
    def _():
        out_r[...] = out_r[...] + r


def _passa_body(n, agg_r, s_r, d_r, b_r, z_r, st_r):
    i = pl.program_id(0)
    sv = _unpack(s_r[...]).astype(jnp.float32)
    zv = d_r[...] * (agg_r[0] + sv) + b_r[...]
    z_r[...] = zv
    gidx = lax.broadcasted_iota(jnp.int32, zv.shape, 0) + i * NBV
    zm = jnp.where(gidx < n, zv, 0.0)
    st = jnp.concatenate([jnp.sum(zm, axis=0, keepdims=True),
                          jnp.sum(zm * zm, axis=0, keepdims=True)])[None]

    @pl.when(i == 0)
    def _():
        st_r[...] = st

    @pl.when(i != 0)
    def _():
        st_r[...] = st_r[...] + st


def _bn_relu(zv, st_r, g_r, be_r, n):
    mu = st_r[0, 0] * (1.0 / n)
    var = st_r[0, 1] * (1.0 / n) - mu * mu
    rstd = lax.rsqrt(var + 1e-5)
    return jnp.maximum((zv - mu) * rstd * g_r[...] + be_r[...], 0.0)


def _passb_body(n, z_r, st_r, g_r, be_r, w_r, d_r, out_r):
    h = _bn_relu(z_r[...], st_r, g_r, be_r, n).astype(jnp.bfloat16)
    m = jnp.dot(h, w_r[...].astype(jnp.bfloat16),
                preferred_element_type=jnp.float32)
    out_r[...] = _pack(d_r[...] * m)


def _bn_only_body(n, z_r, st_r, g_r, be_r, out_r):
    out_r[...] = _bn_relu(z_r[...], st_r, g_r, be_r, n).astype(jnp.bfloat16)


def _head_body(g, h_r, batch_r, pw1_r, pb1_r, pw2_r, pb2_r, pw3_r, pb3_r,
               out_r):
    npad = batch_r.shape[1]
    bvec = batch_r[...]
    gids = lax.broadcasted_iota(jnp.int32, (g, npad), 0)
    p = (gids == jnp.broadcast_to(bvec, (g, npad))).astype(jnp.bfloat16)
    cnt = jnp.sum(p.astype(jnp.float32), axis=1, keepdims=True)
    sums = jnp.dot(p, h_r[...], preferred_element_type=jnp.float32)
    pooled = sums / jnp.maximum(cnt, 1.0)
    o = jnp.maximum(jnp.dot(pooled, pw1_r[...],
                            preferred_element_type=jnp.float32) + pb1_r[...],
                    0.0)
    o = jnp.maximum(jnp.dot(o, pw2_r[...],
                            preferred_element_type=jnp.float32) + pb2_r[...],
                    0.0)
    out_r[...] = jnp.dot(o, pw3_r[...],
                         preferred_element_type=jnp.float32) + pb3_r[...]


def _sc_gather(tab, idx, e_pad, h2):
    per_w = e_pad // NW
    n_ch = per_w // CH
    mesh = plsc.VectorSubcoreMesh(core_axis_name="c", subcore_axis_name="s")

    @functools.partial(
        pl.kernel,
        out_type=jax.ShapeDtypeStruct((e_pad, h2), jnp.int32),
        mesh=mesh,
        scratch_types=(
            [pltpu.VMEM((n_ch, CH), jnp.int32)]
            + [pltpu.VMEM((CH, h2), jnp.int32) for _ in range(KR)]
            + [pltpu.SemaphoreType.DMA for _ in range(2 * KR)]
        ),
    )
    def gather_k(tab_hbm, idx_hbm, out_hbm, idx_all, *scr):
        bufs = scr[:KR]
        gsems = scr[KR:2 * KR]
        osems = scr[2 * KR:]
        wid = lax.axis_index("s") * 2 + lax.axis_index("c")
        base = wid * per_w
        pltpu.sync_copy(idx_hbm.at[pl.ds(wid * n_ch, n_ch)], idx_all)

        def fire(i, b):
            pltpu.async_copy(
                tab_hbm.at[idx_all.at[i]], bufs[b], gsems[b])

        def drain_gather(b):
            pltpu.make_async_copy(
                tab_hbm.at[idx_all.at[0]], bufs[b],
                gsems[b]).wait()

        def flush(i, b):
            pltpu.async_copy(
                bufs[b], out_hbm.at[pl.ds(base + i * CH, CH)], osems[b])

        def drain_flush(b):
            pltpu.make_async_copy(
                bufs[b], out_hbm.at[pl.ds(base, CH)], osems[b]).wait()

        for b in range(KR - 1):
            fire(b, b)

        def grp(gt, carry):
            i = gt * KR
            for b in range(KR):
                cur = i + b
                drain_gather(b)
                flush(cur, b)
                nxt = cur + KR - 1
                bb = (b + KR - 1) % KR

                @pl.when(jnp.logical_and(nxt < n_ch, nxt >= KR))
                def _():
                    drain_flush(bb)
                    fire(nxt, bb)

                @pl.when(jnp.logical_and(nxt < n_ch, nxt < KR))
                def _():
                    fire(nxt, bb)
            return carry

        lax.fori_loop(0, n_ch // KR, grp, 0)
        for b in range(KR):
            drain_flush(b)

    return gather_k(tab, idx.reshape(NW * n_ch, CH))


def kernel(x, edge_index, batch, W1, b1, W2, b2, W3, b3, g1, be1, g2, be2,
           g3, be3, pW1, pb1, pW2, pb2, pW3, pb3):
    n, din = x.shape
    e = edge_index.shape[1]
    h = W1.shape[1]
    h2 = h // 2
    g = 64

    n_pad = ((n + 511) // 512) * 512
    eq = (NW * CH * KR) * EB // math.gcd(NW * CH * KR, EB)
    e_pad = ((e + eq - 1) // eq) * eq
    n_eb = e_pad // EB
    nblk = n_pad // NBV + 1
    s_max = n_eb + 2 * nblk + 8

    src, dst = edge_index[0], edge_index[1]
    order = jnp.argsort(dst)
    src_p = jnp.concatenate(
        [src[order], jnp.full((e_pad - e,), n, jnp.int32)])
    dst_p = jnp.concatenate(
        [dst[order], jnp.full((e_pad - e,), n - 1, jnp.int32)])
    nb_of, eb_of, first, last = _staircase_tables(dst_p, n_eb, nblk, s_max)
    dst3 = dst_p.reshape(n_eb, 1, EB)
    batch_p = jnp.concatenate(
        [batch, jnp.full((n_pad - n,), g, jnp.int32)]).reshape(1, n_pad)
    xp = jnp.pad(x, ((0, n_pad - n), (0, 0)))

    arb = pltpu.CompilerParams(dimension_semantics=("arbitrary",))

    dinv3 = pl.pallas_call(
        functools.partial(_deg_body, n),
        grid_spec=pltpu.PrefetchScalarGridSpec(
            num_scalar_prefetch=4,
            grid=(s_max,),
            in_specs=[pl.BlockSpec((1, 1, EB),
                                   lambda s, nb, ebx, fi, la: (ebx[s], 0, 0))],
            out_specs=pl.BlockSpec((1, 1, NBV),
                                   lambda s, nb, ebx, fi, la: (nb[s], 0, 0)),
        ),
        out_shape=jax.ShapeDtypeStruct((nblk, 1, NBV), jnp.float32),
        compiler_params=arb,
    )(nb_of, eb_of, first, last, dst3)
    dinv = dinv3.reshape(nblk * NBV)[:n_pad].reshape(n_pad, 1)

    def matmul_scale(hmat, w):
        k = hmat.shape[1]
        return pl.pallas_call(
            _matmul_scale_body,
            grid=(n_pad // 256,),
            in_specs=[
                pl.BlockSpec((256, k), lambda i: (i, 0)),
                pl.BlockSpec((k, h), lambda i: (0, 0)),
                pl.BlockSpec((256, 1), lambda i: (i, 0)),
            ],
            out_specs=pl.BlockSpec((256, h2), lambda i: (i, 0)),
            out_shape=jax.ShapeDtypeStruct((n_pad, h2), jnp.int32),
        )(hmat, w, dinv)

    def seg_sum(ev):
        ev3 = ev.reshape(n_eb, EB, h2)
        return pl.pallas_call(
            _seg_body,
            grid_spec=pltpu.PrefetchScalarGridSpec(
                num_scalar_prefetch=3,
                grid=(s_max,),
                in_specs=[
                    pl.BlockSpec((1, 1, EB),
                                 lambda s, nb, ebx, fi: (ebx[s], 0, 0)),
                    pl.BlockSpec((1, EB, h2),
                                 lambda s, nb, ebx, fi: (ebx[s], 0, 0)),
                ],
                out_specs=pl.BlockSpec((1, NBV, h),
                                       lambda s, nb, ebx, fi: (nb[s], 0, 0)),
            ),
            out_shape=jax.ShapeDtypeStruct((nblk, NBV, h), jnp.float32),
            compiler_params=arb,
        )(nb_of, eb_of, first, dst3, ev3)

    def pass_a(agg, sarr, b):
        return pl.pallas_call(
            functools.partial(_passa_body, n),
            grid=(n_pad // NBV,),
            in_specs=[
                pl.BlockSpec((1, NBV, h), lambda i: (i, 0, 0)),
                pl.BlockSpec((NBV, h2), lambda i: (i, 0)),
                pl.BlockSpec((NBV, 1), lambda i: (i, 0)),
                pl.BlockSpec((1, h), lambda i: (0, 0)),
            ],
            out_specs=[
                pl.BlockSpec((NBV, h), lambda i: (i, 0)),
                pl.BlockSpec((1, 2, h), lambda i: (0, 0, 0)),
            ],
            out_shape=[
                jax.ShapeDtypeStruct((n_pad, h), jnp.float32),
                jax.ShapeDtypeStruct((1, 2, h), jnp.float32),
            ],
            compiler_params=arb,
        )(agg, sarr, dinv, b.reshape(1, h))

    def pass_b(z, st, gg, be, w):
        return pl.pallas_call(
            functools.partial(_passb_body, n),
            grid=(n_pad // 256,),
            in_specs=[
                pl.BlockSpec((256, h), lambda i: (i, 0)),
                pl.BlockSpec((1, 2, h), lambda i: (0, 0, 0)),
                pl.BlockSpec((1, h), lambda i: (0, 0)),
                pl.BlockSpec((1, h), lambda i: (0, 0)),
                pl.BlockSpec((h, h), lambda i: (0, 0)),
                pl.BlockSpec((256, 1), lambda i: (i, 0)),
            ],
            out_specs=pl.BlockSpec((256, h2), lambda i: (i, 0)),
            out_shape=jax.ShapeDtypeStruct((n_pad, h2), jnp.int32),
        )(z, st, gg.reshape(1, h), be.reshape(1, h), w, dinv)

    s1 = matmul_scale(xp, W1)
    z1, st1 = pass_a(seg_sum(_sc_gather(s1, src_p, e_pad, h2)), s1, b1)

    s2 = pass_b(z1, st1, g1, be1, W2)
    z2, st2 = pass_a(seg_sum(_sc_gather(s2, src_p, e_pad, h2)), s2, b2)

    s3 = pass_b(z2, st2, g2, be2, W3)
    z3, st3 = pass_a(seg_sum(_sc_gather(s3, src_p, e_pad, h2)), s3, b3)

    h3 = pl.pallas_call(
        functools.partial(_bn_only_body, n),
        grid=(n_pad // 256,),
        in_specs=[
            pl.BlockSpec((256, h), lambda i: (i, 0)),
            pl.BlockSpec((1, 2, h), lambda i: (0, 0, 0)),
            pl.BlockSpec((1, h), lambda i: (0, 0)),
            pl.BlockSpec((1, h), lambda i: (0, 0)),
        ],
        out_specs=pl.BlockSpec((256, h), lambda i: (i, 0)),
        out_shape=jax.ShapeDtypeStruct((n_pad, h), jnp.bfloat16),
    )(z3, st3, g3.reshape(1, h), be3.reshape(1, h))

    out = pl.pallas_call(
        functools.partial(_head_body, g),
        in_specs=[pl.BlockSpec(sh, functools.partial(lambda s: (0,) * len(s),
                                                     sh))
                  for sh in [(n_pad, h), (1, n_pad), (h, h // 2), (1, h // 2),
                             (h // 2, h // 4), (1, h // 4), (h // 4, 1),
                             (1, 1)]],
        out_specs=pl.BlockSpec((g, 1), lambda: (0, 0)),
        out_shape=jax.ShapeDtypeStruct((g, 1), jnp.float32),
    )(h3, batch_p, pW1, pb1.reshape(1, h // 2), pW2, pb2.reshape(1, h // 4),
      pW3, pb3.reshape(1, 1))
    return out


# speedup vs baseline: 1.1188x; 1.1188x over previous
import functools
import math

import jax
import jax.numpy as jnp
from jax import lax
from jax.experimental import pallas as pl
from jax.experimental.pallas import tpu as pltpu
from jax.experimental.pallas import tpu_sc as plsc

EB = 1024
NBV = 128
NW = 32
CH = 80
KR = 4


def _pack(m):
    c2 = m.shape[1] // 2
    mb = m.astype(jnp.bfloat16).reshape(m.shape[0], 2, c2)
    return pltpu.bitcast(mb, jnp.int32).reshape(m.shape[0], c2)


def _unpack(p):
    b = pltpu.bitcast(p.reshape(p.shape[0], 1, p.shape[1]), jnp.bfloat16)
    return b.reshape(p.shape[0], 2 * p.shape[1])


def _staircase_tables(dst_p, n_eb, nblk, s_max):
    lo = dst_p[0::EB] // NBV
    hi = dst_p[EB - 1::EB] // NBV
    nbs = jnp.arange(nblk, dtype=jnp.int32)
    ebf = jnp.searchsorted(hi, nbs, side="left").astype(jnp.int32)
    ebl = (jnp.searchsorted(lo, nbs, side="right") - 1).astype(jnp.int32)
    cnt = jnp.maximum(ebl - ebf + 1, 1)
    offs = jnp.concatenate([jnp.zeros((1,), jnp.int32),
                            jnp.cumsum(cnt)[:-1].astype(jnp.int32)])
    jj = jnp.arange(s_max, dtype=jnp.int32)
    nb_of = (jnp.searchsorted(offs, jj, side="right") - 1).astype(jnp.int32)
    within = jj - offs[nb_of]
    eb_of = jnp.clip(ebf[nb_of] + within, 0, n_eb - 1).astype(jnp.int32)
    first = (within == 0).astype(jnp.int32)
    last = (within == (cnt[nb_of] - 1)).astype(jnp.int32)
    total = offs[nblk - 1] + cnt[nblk - 1]
    live = (jj < total).astype(jnp.int32)
    return nb_of, eb_of, first, last, live


def _deg_body(n, nb_r, eb_r, fi_r, la_r, li_r, dst_r, out_r):
    s = pl.program_id(0)
    base = nb_r[s] * NBV

    @pl.when(li_r[s] == 1)
    def _():
        dstv = dst_r[0]
        rows = lax.broadcasted_iota(jnp.int32, (NBV, EB), 0) + base
        t = (rows == jnp.broadcast_to(dstv, (NBV, EB))).astype(jnp.float32)
        c = jnp.sum(t, axis=1).reshape(1, 1, NBV)

        @pl.when(fi_r[s] == 1)
        def _():
            out_r[...] = c

        @pl.when(fi_r[s] == 0)
        def _():
            out_r[...] = out_r[...] + c

        @pl.when(la_r[s] == 1)
        def _():
            gidx = lax.broadcasted_iota(jnp.int32, (1, 1, NBV), 2) + base
            out_r[...] = jnp.where(gidx < n,
                                   lax.rsqrt(out_r[...] + 1.0), 0.0)


def _matmul_scale_body(x_r, w_r, d_r, out_r):
    m = jnp.dot(x_r[...].astype(jnp.bfloat16), w_r[...].astype(jnp.bfloat16),
                preferred_element_type=jnp.float32)
    out_r[...] = _pack(d_r[...] * m)


def _seg_body(nb_r, eb_r, fi_r, li_r, dst_r, ev_r, out_r):
    s = pl.program_id(0)
    base = nb_r[s] * NBV

    @pl.when(li_r[s] == 1)
    def _():
        dstv = dst_r[0]
        rows = lax.broadcasted_iota(jnp.int32, (NBV, EB), 0) + base
        t = (rows == jnp.broadcast_to(dstv, (NBV, EB))).astype(jnp.bfloat16)
        evb = _unpack(ev_r[0])
        r = jnp.dot(t, evb, preferred_element_type=jnp.float32)[None]

        @pl.when(fi_r[s] == 1)
        def _():
            out_r[...] = r

        @pl.when(fi_r[s] == 0)
        def _():
            out_r[...] = out_r[...] + r


def _passa_body(n, agg_r, s_r, d_r, b_r, z_r, st_r):
    i = pl.program_id(0)
    sv = _unpack(s_r[...]).astype(jnp.float32)
    zv = d_r[...] * (agg_r[0] + sv) + b_r[...]
    z_r[...] = zv
    gidx = lax.broadcasted_iota(jnp.int32, zv.shape, 0) + i * NBV
    zm = jnp.where(gidx < n, zv, 0.0)
    st = jnp.concatenate([jnp.sum(zm, axis=0, keepdims=True),
                          jnp.sum(zm * zm, axis=0, keepdims=True)])[None]

    @pl.when(i == 0)
    def _():
        st_r[...] = st

    @pl.when(i != 0)
    def _():
        st_r[...] = st_r[...] + st


def _bn_relu(zv, st_r, g_r, be_r, n):
    mu = st_r[0, 0] * (1.0 / n)
    var = st_r[0, 1] * (1.0 / n) - mu * mu
    rstd = lax.rsqrt(var + 1e-5)
    return jnp.maximum((zv - mu) * rstd * g_r[...] + be_r[...], 0.0)


def _passb_body(n, z_r, st_r, g_r, be_r, w_r, d_r, out_r):
    h = _bn_relu(z_r[...], st_r, g_r, be_r, n).astype(jnp.bfloat16)
    m = jnp.dot(h, w_r[...].astype(jnp.bfloat16),
                preferred_element_type=jnp.float32)
    out_r[...] = _pack(d_r[...] * m)


def _bn_only_body(n, z_r, st_r, g_r, be_r, out_r):
    out_r[...] = _bn_relu(z_r[...], st_r, g_r, be_r, n).astype(jnp.bfloat16)


def _head_body(g, h_r, batch_r, pw1_r, pb1_r, pw2_r, pb2_r, pw3_r, pb3_r,
               out_r):
    npad = batch_r.shape[1]
    bvec = batch_r[...]
    gids = lax.broadcasted_iota(jnp.int32, (g, npad), 0)
    p = (gids == jnp.broadcast_to(bvec, (g, npad))).astype(jnp.bfloat16)
    cnt = jnp.sum(p.astype(jnp.float32), axis=1, keepdims=True)
    sums = jnp.dot(p, h_r[...], preferred_element_type=jnp.float32)
    pooled = sums / jnp.maximum(cnt, 1.0)
    o = jnp.maximum(jnp.dot(pooled, pw1_r[...],
                            preferred_element_type=jnp.float32) + pb1_r[...],
                    0.0)
    o = jnp.maximum(jnp.dot(o, pw2_r[...],
                            preferred_element_type=jnp.float32) + pb2_r[...],
                    0.0)
    out_r[...] = jnp.dot(o, pw3_r[...],
                         preferred_element_type=jnp.float32) + pb3_r[...]


def _sc_gather(tab, idx, e_pad, h2):
    per_w = e_pad // NW
    n_ch = per_w // CH
    mesh = plsc.VectorSubcoreMesh(core_axis_name="c", subcore_axis_name="s")

    @functools.partial(
        pl.kernel,
        out_type=jax.ShapeDtypeStruct((e_pad, h2), jnp.int32),
        mesh=mesh,
        scratch_types=(
            [pltpu.VMEM((n_ch, CH), jnp.int32)]
            + [pltpu.VMEM((CH, h2), jnp.int32) for _ in range(KR)]
            + [pltpu.SemaphoreType.DMA for _ in range(2 * KR)]
        ),
    )
    def gather_k(tab_hbm, idx_hbm, out_hbm, idx_all, *scr):
        bufs = scr[:KR]
        gsems = scr[KR:2 * KR]
        osems = scr[2 * KR:]
        wid = lax.axis_index("s") * 2 + lax.axis_index("c")
        base = wid * per_w
        pltpu.sync_copy(idx_hbm.at[pl.ds(wid * n_ch, n_ch)], idx_all)

        def fire(i, b):
            pltpu.async_copy(
                tab_hbm.at[idx_all.at[i]], bufs[b], gsems[b])

        def drain_gather(b):
            pltpu.make_async_copy(
                tab_hbm.at[idx_all.at[0]], bufs[b],
                gsems[b]).wait()

        def flush(i, b):
            pltpu.async_copy(
                bufs[b], out_hbm.at[pl.ds(base + i * CH, CH)], osems[b])

        def drain_flush(b):
            pltpu.make_async_copy(
                bufs[b], out_hbm.at[pl.ds(base, CH)], osems[b]).wait()

        for b in range(KR - 1):
            fire(b, b)

        def grp(gt, carry):
            i = gt * KR
            for b in range(KR):
                cur = i + b
                drain_gather(b)
                flush(cur, b)
                nxt = cur + KR - 1
                bb = (b + KR - 1) % KR

                @pl.when(jnp.logical_and(nxt < n_ch, nxt >= KR))
                def _():
                    drain_flush(bb)
                    fire(nxt, bb)

                @pl.when(jnp.logical_and(nxt < n_ch, nxt < KR))
                def _():
                    fire(nxt, bb)
            return carry

        lax.fori_loop(0, n_ch // KR, grp, 0)
        for b in range(KR):
            drain_flush(b)

    return gather_k(tab, idx.reshape(NW * n_ch, CH))


def kernel(x, edge_index, batch, W1, b1, W2, b2, W3, b3, g1, be1, g2, be2,
           g3, be3, pW1, pb1, pW2, pb2, pW3, pb3):
    n, din = x.shape
    e = edge_index.shape[1]
    h = W1.shape[1]
    h2 = h // 2
    g = 64

    n_pad = ((n + 511) // 512) * 512
    eq = (NW * CH * KR) * EB // math.gcd(NW * CH * KR, EB)
    e_pad = ((e + eq - 1) // eq) * eq
    n_eb = e_pad // EB
    nblk = n_pad // NBV + 1
    s_max = n_eb + 2 * nblk + 8

    src, dst = edge_index[0], edge_index[1]
    order = jnp.argsort(dst)
    src_p = jnp.concatenate(
        [src[order], jnp.full((e_pad - e,), n, jnp.int32)])
    dst_p = jnp.concatenate(
        [dst[order], jnp.full((e_pad - e,), n - 1, jnp.int32)])
    nb_of, eb_of, first, last, live = _staircase_tables(
        dst_p, n_eb, nblk, s_max)
    dst3 = dst_p.reshape(n_eb, 1, EB)
    batch_p = jnp.concatenate(
        [batch, jnp.full((n_pad - n,), g, jnp.int32)]).reshape(1, n_pad)
    xp = jnp.pad(x, ((0, n_pad - n), (0, 0)))

    arb = pltpu.CompilerParams(dimension_semantics=("arbitrary",))

    dinv3 = pl.pallas_call(
        functools.partial(_deg_body, n),
        grid_spec=pltpu.PrefetchScalarGridSpec(
            num_scalar_prefetch=5,
            grid=(s_max,),
            in_specs=[pl.BlockSpec(
                (1, 1, EB),
                lambda s, nb, ebx, fi, la, li: (ebx[s], 0, 0))],
            out_specs=pl.BlockSpec(
                (1, 1, NBV),
                lambda s, nb, ebx, fi, la, li: (nb[s], 0, 0)),
        ),
        out_shape=jax.ShapeDtypeStruct((nblk, 1, NBV), jnp.float32),
        compiler_params=arb,
    )(nb_of, eb_of, first, last, live, dst3)
    dinv = dinv3.reshape(nblk * NBV)[:n_pad].reshape(n_pad, 1)

    def matmul_scale(hmat, w):
        k = hmat.shape[1]
        return pl.pallas_call(
            _matmul_scale_body,
            grid=(n_pad // 256,),
            in_specs=[
                pl.BlockSpec((256, k), lambda i: (i, 0)),
                pl.BlockSpec((k, h), lambda i: (0, 0)),
                pl.BlockSpec((256, 1), lambda i: (i, 0)),
            ],
            out_specs=pl.BlockSpec((256, h2), lambda i: (i, 0)),
            out_shape=jax.ShapeDtypeStruct((n_pad, h2), jnp.int32),
        )(hmat, w, dinv)

    def seg_sum(ev):
        ev3 = ev.reshape(n_eb, EB, h2)
        return pl.pallas_call(
            _seg_body,
            grid_spec=pltpu.PrefetchScalarGridSpec(
                num_scalar_prefetch=4,
                grid=(s_max,),
                in_specs=[
                    pl.BlockSpec(
                        (1, 1, EB),
                        lambda s, nb, ebx, fi, li: (ebx[s], 0, 0)),
                    pl.BlockSpec(
                        (1, EB, h2),
                        lambda s, nb, ebx, fi, li: (ebx[s], 0, 0)),
                ],
                out_specs=pl.BlockSpec(
                    (1, NBV, h),
                    lambda s, nb, ebx, fi, li: (nb[s], 0, 0)),
            ),
            out_shape=jax.ShapeDtypeStruct((nblk, NBV, h), jnp.float32),
            compiler_params=arb,
        )(nb_of, eb_of, first, live, dst3, ev3)

    def pass_a(agg, sarr, b):
        return pl.pallas_call(
            functools.partial(_passa_body, n),
            grid=(n_pad // NBV,),
            in_specs=[
                pl.BlockSpec((1, NBV, h), lambda i: (i, 0, 0)),
                pl.BlockSpec((NBV, h2), lambda i: (i, 0)),
                pl.BlockSpec((NBV, 1), lambda i: (i, 0)),
                pl.BlockSpec((1, h), lambda i: (0, 0)),
            ],
            out_specs=[
                pl.BlockSpec((NBV, h), lambda i: (i, 0)),
                pl.BlockSpec((1, 2, h), lambda i: (0, 0, 0)),
            ],
            out_shape=[
                jax.ShapeDtypeStruct((n_pad, h), jnp.float32),
                jax.ShapeDtypeStruct((1, 2, h), jnp.float32),
            ],
            compiler_params=arb,
        )(agg, sarr, dinv, b.reshape(1, h))

    def pass_b(z, st, gg, be, w):
        return pl.pallas_call(
            functools.partial(_passb_body, n),
            grid=(n_pad // 256,),
            in_specs=[
                pl.BlockSpec((256, h), lambda i: (i, 0)),
                pl.BlockSpec((1, 2, h), lambda i: (0, 0, 0)),
                pl.BlockSpec((1, h), lambda i: (0, 0)),
                pl.BlockSpec((1, h), lambda i: (0, 0)),
                pl.BlockSpec((h, h), lambda i: (0, 0)),
                pl.BlockSpec((256, 1), lambda i: (i, 0)),
            ],
            out_specs=pl.BlockSpec((256, h2), lambda i: (i, 0)),
            out_shape=jax.ShapeDtypeStruct((n_pad, h2), jnp.int32),
        )(z, st, gg.reshape(1, h), be.reshape(1, h), w, dinv)

    s1 = matmul_scale(xp, W1)
    z1, st1 = pass_a(seg_sum(_sc_gather(s1, src_p, e_pad, h2)), s1, b1)

    s2 = pass_b(z1, st1, g1, be1, W2)
    z2, st2 = pass_a(seg_sum(_sc_gather(s2, src_p, e_pad, h2)), s2, b2)

    s3 = pass_b(z2, st2, g2, be2, W3)
    z3, st3 = pass_a(seg_sum(_sc_gather(s3, src_p, e_pad, h2)), s3, b3)

    h3 = pl.pallas_call(
        functools.partial(_bn_only_body, n),
        grid=(n_pad // 256,),
        in_specs=[
            pl.BlockSpec((256, h), lambda i: (i, 0)),
            pl.BlockSpec((1, 2, h), lambda i: (0, 0, 0)),
            pl.BlockSpec((1, h), lambda i: (0, 0)),
            pl.BlockSpec((1, h), lambda i: (0, 0)),
        ],
        out_specs=pl.BlockSpec((256, h), lambda i: (i, 0)),
        out_shape=jax.ShapeDtypeStruct((n_pad, h), jnp.bfloat16),
    )(z3, st3, g3.reshape(1, h), be3.reshape(1, h))

    out = pl.pallas_call(
        functools.partial(_head_body, g),
        in_specs=[pl.BlockSpec(sh, functools.partial(lambda s: (0,) * len(s),
                                                     sh))
                  for sh in [(n_pad, h), (1, n_pad), (h, h // 2), (1, h // 2),
                             (h // 2, h // 4), (1, h // 4), (h // 4, 1),
                             (1, 1)]],
        out_specs=pl.BlockSpec((g, 1), lambda: (0, 0)),
        out_shape=jax.ShapeDtypeStruct((g, 1), jnp.float32),
    )(h3, batch_p, pW1, pb1.reshape(1, h // 2), pW2, pb2.reshape(1, h // 4),
      pW3, pb3.reshape(1, 1))
    return out


# speedup vs baseline: 1.1843x; 1.0585x over previous
import functools
import math

import jax
import jax.numpy as jnp
from jax import lax
from jax.experimental import pallas as pl
from jax.experimental.pallas import tpu as pltpu
from jax.experimental.pallas import tpu_sc as plsc

EB = 1024
NBV = 128
NW = 32
CH = 80
KR = 4


def _pack(m):
    c2 = m.shape[1] // 2
    mb = m.astype(jnp.bfloat16).reshape(m.shape[0], 2, c2)
    return pltpu.bitcast(mb, jnp.int32).reshape(m.shape[0], c2)


def _unpack(p):
    b = pltpu.bitcast(p.reshape(p.shape[0], 1, p.shape[1]), jnp.bfloat16)
    return b.reshape(p.shape[0], 2 * p.shape[1])


def _staircase_tables(dst_p, n_eb, nblk, s_max):
    lo = dst_p[0::EB] // NBV
    hi = dst_p[EB - 1::EB] // NBV
    nbs = jnp.arange(nblk, dtype=jnp.int32)
    ebf = jnp.searchsorted(hi, nbs, side="left").astype(jnp.int32)
    ebl = (jnp.searchsorted(lo, nbs, side="right") - 1).astype(jnp.int32)
    cnt = jnp.maximum(ebl - ebf + 1, 1)
    offs = jnp.concatenate([jnp.zeros((1,), jnp.int32),
                            jnp.cumsum(cnt)[:-1].astype(jnp.int32)])
    jj = jnp.arange(s_max, dtype=jnp.int32)
    nb_of = (jnp.searchsorted(offs, jj, side="right") - 1).astype(jnp.int32)
    within = jj - offs[nb_of]
    eb_of = jnp.clip(ebf[nb_of] + within, 0, n_eb - 1).astype(jnp.int32)
    first = (within == 0).astype(jnp.int32)
    last = (within == (cnt[nb_of] - 1)).astype(jnp.int32)
    total = offs[nblk - 1] + cnt[nblk - 1]
    live = (jj < total).astype(jnp.int32)
    return nb_of, eb_of, first, last, live


def _deg_body(n, nb_r, eb_r, fi_r, la_r, li_r, dst_r, out_r):
    s = pl.program_id(0)
    base = nb_r[s] * NBV

    @pl.when(li_r[s] == 1)
    def _():
        dstv = dst_r[0]
        rows = lax.broadcasted_iota(jnp.int32, (NBV, EB), 0) + base
        t = (rows == jnp.broadcast_to(dstv, (NBV, EB))).astype(jnp.float32)
        c = jnp.sum(t, axis=1).reshape(1, 1, NBV)

        @pl.when(fi_r[s] == 1)
        def _():
            out_r[...] = c

        @pl.when(fi_r[s] == 0)
        def _():
            out_r[...] = out_r[...] + c

        @pl.when(la_r[s] == 1)
        def _():
            gidx = lax.broadcasted_iota(jnp.int32, (1, 1, NBV), 2) + base
            out_r[...] = jnp.where(gidx < n,
                                   lax.rsqrt(out_r[...] + 1.0), 0.0)


def _matmul_scale_body(x_r, w_r, d_r, out_r):
    m = jnp.dot(x_r[...].astype(jnp.bfloat16), w_r[...].astype(jnp.bfloat16),
                preferred_element_type=jnp.float32)
    out_r[...] = _pack(d_r[...] * m)


def _seg_body(nb_r, eb_r, fi_r, li_r, dst_r, ev_r, out_r):
    s = pl.program_id(0)
    base = nb_r[s] * NBV

    @pl.when(li_r[s] == 1)
    def _():
        dstv = dst_r[0]
        rows = lax.broadcasted_iota(jnp.int32, (NBV, EB), 0) + base
        t = (rows == jnp.broadcast_to(dstv, (NBV, EB))).astype(jnp.bfloat16)
        evb = _unpack(ev_r[0])
        r = jnp.dot(t, evb, preferred_element_type=jnp.float32)[None]

        @pl.when(fi_r[s] == 1)
        def _():
            out_r[...] = r

        @pl.when(fi_r[s] == 0)
        def _():
            out_r[...] = out_r[...] + r


def _passa_body(n, agg_r, s_r, d_r, b_r, z_r, st_r):
    i = pl.program_id(0)
    sv = _unpack(s_r[...]).astype(jnp.float32)
    zv = d_r[...] * (agg_r[0] + sv) + b_r[...]
    z_r[...] = zv
    gidx = lax.broadcasted_iota(jnp.int32, zv.shape, 0) + i * NBV
    zm = jnp.where(gidx < n, zv, 0.0)
    st = jnp.concatenate([jnp.sum(zm, axis=0, keepdims=True),
                          jnp.sum(zm * zm, axis=0, keepdims=True)])[None]

    @pl.when(i == 0)
    def _():
        st_r[...] = st

    @pl.when(i != 0)
    def _():
        st_r[...] = st_r[...] + st


def _bn_relu(zv, st_r, g_r, be_r, n):
    mu = st_r[0, 0] * (1.0 / n)
    var = st_r[0, 1] * (1.0 / n) - mu * mu
    rstd = lax.rsqrt(var + 1e-5)
    return jnp.maximum((zv - mu) * rstd * g_r[...] + be_r[...], 0.0)


def _passb_body(n, z_r, st_r, g_r, be_r, w_r, d_r, out_r):
    h = _bn_relu(z_r[...], st_r, g_r, be_r, n).astype(jnp.bfloat16)
    m = jnp.dot(h, w_r[...].astype(jnp.bfloat16),
                preferred_element_type=jnp.float32)
    out_r[...] = _pack(d_r[...] * m)


def _bn_only_body(n, z_r, st_r, g_r, be_r, out_r):
    out_r[...] = _bn_relu(z_r[...], st_r, g_r, be_r, n).astype(jnp.bfloat16)


def _head_body(g, h_r, batch_r, pw1_r, pb1_r, pw2_r, pb2_r, pw3_r, pb3_r,
               out_r):
    npad = batch_r.shape[1]
    bvec = batch_r[...]
    gids = lax.broadcasted_iota(jnp.int32, (g, npad), 0)
    p = (gids == jnp.broadcast_to(bvec, (g, npad))).astype(jnp.bfloat16)
    cnt = jnp.sum(p.astype(jnp.float32), axis=1, keepdims=True)
    sums = jnp.dot(p, h_r[...], preferred_element_type=jnp.float32)
    pooled = sums / jnp.maximum(cnt, 1.0)
    o = jnp.maximum(jnp.dot(pooled, pw1_r[...],
                            preferred_element_type=jnp.float32) + pb1_r[...],
                    0.0)
    o = jnp.maximum(jnp.dot(o, pw2_r[...],
                            preferred_element_type=jnp.float32) + pb2_r[...],
                    0.0)
    out_r[...] = jnp.dot(o, pw3_r[...],
                         preferred_element_type=jnp.float32) + pb3_r[...]


def _sc_gather(tab, idx, e_pad, h2):
    per_w = e_pad // NW
    n_ch = per_w // CH
    mesh = plsc.VectorSubcoreMesh(core_axis_name="c", subcore_axis_name="s")

    @functools.partial(
        pl.kernel,
        out_type=jax.ShapeDtypeStruct((e_pad, h2), jnp.int32),
        mesh=mesh,
        scratch_types=(
            [pltpu.VMEM((n_ch, CH), jnp.int32)]
            + [pltpu.VMEM((CH, h2), jnp.int32) for _ in range(KR)]
            + [pltpu.SemaphoreType.DMA for _ in range(2 * KR)]
        ),
    )
    def gather_k(tab_hbm, idx_hbm, out_hbm, idx_all, *scr):
        bufs = scr[:KR]
        gsems = scr[KR:2 * KR]
        osems = scr[2 * KR:]
        wid = lax.axis_index("s") * 2 + lax.axis_index("c")
        base = wid * per_w
        pltpu.sync_copy(idx_hbm.at[pl.ds(wid * n_ch, n_ch)], idx_all)

        def fire(i, b):
            pltpu.async_copy(
                tab_hbm.at[idx_all.at[i]], bufs[b], gsems[b])

        def drain_gather(b):
            pltpu.make_async_copy(
                tab_hbm.at[idx_all.at[0]], bufs[b],
                gsems[b]).wait()

        def flush(i, b):
            pltpu.async_copy(
                bufs[b], out_hbm.at[pl.ds(base + i * CH, CH)], osems[b])

        def drain_flush(b):
            pltpu.make_async_copy(
                bufs[b], out_hbm.at[pl.ds(base, CH)], osems[b]).wait()

        for b in range(KR - 1):
            fire(b, b)

        def grp(gt, carry):
            i = gt * KR
            for b in range(KR):
                cur = i + b
                drain_gather(b)
                flush(cur, b)
                nxt = cur + KR - 1
                bb = (b + KR - 1) % KR

                @pl.when(jnp.logical_and(nxt < n_ch, nxt >= KR))
                def _():
                    drain_flush(bb)
                    fire(nxt, bb)

                @pl.when(jnp.logical_and(nxt < n_ch, nxt < KR))
                def _():
                    fire(nxt, bb)
            return carry

        lax.fori_loop(0, n_ch // KR, grp, 0)
        for b in range(KR):
            drain_flush(b)

    return gather_k(tab, idx.reshape(NW * n_ch, CH))


def kernel(x, edge_index, batch, W1, b1, W2, b2, W3, b3, g1, be1, g2, be2,
           g3, be3, pW1, pb1, pW2, pb2, pW3, pb3):
    n, din = x.shape
    e = edge_index.shape[1]
    h = W1.shape[1]
    h2 = h // 2
    g = 64

    n_pad = ((n + 511) // 512) * 512
    eq = (NW * CH * KR) * EB // math.gcd(NW * CH * KR, EB)
    e_pad = ((e + eq - 1) // eq) * eq
    n_eb = e_pad // EB
    nblk = n_pad // NBV + 1
    s_max = n_eb + 2 * nblk + 8

    src, dst = edge_index[0], edge_index[1]
    comp = jnp.sort((dst.astype(jnp.uint32) << 18)
                    | jnp.arange(e, dtype=jnp.uint32))
    order = (comp & jnp.uint32((1 << 18) - 1)).astype(jnp.int32)
    src_p = jnp.concatenate(
        [src[order], jnp.full((e_pad - e,), n, jnp.int32)])
    dst_p = jnp.concatenate(
        [(comp >> 18).astype(jnp.int32),
         jnp.full((e_pad - e,), n - 1, jnp.int32)])
    nb_of, eb_of, first, last, live = _staircase_tables(
        dst_p, n_eb, nblk, s_max)
    dst3 = dst_p.reshape(n_eb, 1, EB)
    batch_p = jnp.concatenate(
        [batch, jnp.full((n_pad - n,), g, jnp.int32)]).reshape(1, n_pad)
    xp = jnp.pad(x, ((0, n_pad - n), (0, 0)))

    arb = pltpu.CompilerParams(dimension_semantics=("arbitrary",))

    dinv3 = pl.pallas_call(
        functools.partial(_deg_body, n),
        grid_spec=pltpu.PrefetchScalarGridSpec(
            num_scalar_prefetch=5,
            grid=(s_max,),
            in_specs=[pl.BlockSpec(
                (1, 1, EB),
                lambda s, nb, ebx, fi, la, li: (ebx[s], 0, 0))],
            out_specs=pl.BlockSpec(
                (1, 1, NBV),
                lambda s, nb, ebx, fi, la, li: (nb[s], 0, 0)),
        ),
        out_shape=jax.ShapeDtypeStruct((nblk, 1, NBV), jnp.float32),
        compiler_params=arb,
    )(nb_of, eb_of, first, last, live, dst3)
    dinv = dinv3.reshape(nblk * NBV)[:n_pad].reshape(n_pad, 1)

    def matmul_scale(hmat, w):
        k = hmat.shape[1]
        return pl.pallas_call(
            _matmul_scale_body,
            grid=(n_pad // 256,),
            in_specs=[
                pl.BlockSpec((256, k), lambda i: (i, 0)),
                pl.BlockSpec((k, h), lambda i: (0, 0)),
                pl.BlockSpec((256, 1), lambda i: (i, 0)),
            ],
            out_specs=pl.BlockSpec((256, h2), lambda i: (i, 0)),
            out_shape=jax.ShapeDtypeStruct((n_pad, h2), jnp.int32),
        )(hmat, w, dinv)

    def seg_sum(ev):
        ev3 = ev.reshape(n_eb, EB, h2)
        return pl.pallas_call(
            _seg_body,
            grid_spec=pltpu.PrefetchScalarGridSpec(
                num_scalar_prefetch=4,
                grid=(s_max,),
                in_specs=[
                    pl.BlockSpec(
                        (1, 1, EB),
                        lambda s, nb, ebx, fi, li: (ebx[s], 0, 0)),
                    pl.BlockSpec(
                        (1, EB, h2),
                        lambda s, nb, ebx, fi, li: (ebx[s], 0, 0)),
                ],
                out_specs=pl.BlockSpec(
                    (1, NBV, h),
                    lambda s, nb, ebx, fi, li: (nb[s], 0, 0)),
            ),
            out_shape=jax.ShapeDtypeStruct((nblk, NBV, h), jnp.float32),
            compiler_params=arb,
        )(nb_of, eb_of, first, live, dst3, ev3)

    def pass_a(agg, sarr, b):
        return pl.pallas_call(
            functools.partial(_passa_body, n),
            grid=(n_pad // NBV,),
            in_specs=[
                pl.BlockSpec((1, NBV, h), lambda i: (i, 0, 0)),
                pl.BlockSpec((NBV, h2), lambda i: (i, 0)),
                pl.BlockSpec((NBV, 1), lambda i: (i, 0)),
                pl.BlockSpec((1, h), lambda i: (0, 0)),
            ],
            out_specs=[
                pl.BlockSpec((NBV, h), lambda i: (i, 0)),
                pl.BlockSpec((1, 2, h), lambda i: (0, 0, 0)),
            ],
            out_shape=[
                jax.ShapeDtypeStruct((n_pad, h), jnp.float32),
                jax.ShapeDtypeStruct((1, 2, h), jnp.float32),
            ],
            compiler_params=arb,
        )(agg, sarr, dinv, b.reshape(1, h))

    def pass_b(z, st, gg, be, w):
        return pl.pallas_call(
            functools.partial(_passb_body, n),
            grid=(n_pad // 256,),
            in_specs=[
                pl.BlockSpec((256, h), lambda i: (i, 0)),
                pl.BlockSpec((1, 2, h), lambda i: (0, 0, 0)),
                pl.BlockSpec((1, h), lambda i: (0, 0)),
                pl.BlockSpec((1, h), lambda i: (0, 0)),
                pl.BlockSpec((h, h), lambda i: (0, 0)),
                pl.BlockSpec((256, 1), lambda i: (i, 0)),
            ],
            out_specs=pl.BlockSpec((256, h2), lambda i: (i, 0)),
            out_shape=jax.ShapeDtypeStruct((n_pad, h2), jnp.int32),
        )(z, st, gg.reshape(1, h), be.reshape(1, h), w, dinv)

    s1 = matmul_scale(xp, W1)
    z1, st1 = pass_a(seg_sum(_sc_gather(s1, src_p, e_pad, h2)), s1, b1)

    s2 = pass_b(z1, st1, g1, be1, W2)
    z2, st2 = pass_a(seg_sum(_sc_gather(s2, src_p, e_pad, h2)), s2, b2)

    s3 = pass_b(z2, st2, g2, be2, W3)
    z3, st3 = pass_a(seg_sum(_sc_gather(s3, src_p, e_pad, h2)), s3, b3)

    h3 = pl.pallas_call(
        functools.partial(_bn_only_body, n),
        grid=(n_pad // 256,),
        in_specs=[
            pl.BlockSpec((256, h), lambda i: (i, 0)),
            pl.BlockSpec((1, 2, h), lambda i: (0, 0, 0)),
            pl.BlockSpec((1, h), lambda i: (0, 0)),
            pl.BlockSpec((1, h), lambda i: (0, 0)),
        ],
        out_specs=pl.BlockSpec((256, h), lambda i: (i, 0)),
        out_shape=jax.ShapeDtypeStruct((n_pad, h), jnp.bfloat16),
    )(z3, st3, g3.reshape(1, h), be3.reshape(1, h))

    out = pl.pallas_call(
        functools.partial(_head_body, g),
        in_specs=[pl.BlockSpec(sh, functools.partial(lambda s: (0,) * len(s),
                                                     sh))
                  for sh in [(n_pad, h), (1, n_pad), (h, h // 2), (1, h // 2),
                             (h // 2, h // 4), (1, h // 4), (h // 4, 1),
                             (1, 1)]],
        out_specs=pl.BlockSpec((g, 1), lambda: (0, 0)),
        out_shape=jax.ShapeDtypeStruct((g, 1), jnp.float32),
    )(h3, batch_p, pW1, pb1.reshape(1, h // 2), pW2, pb2.reshape(1, h // 4),
      pW3, pb3.reshape(1, 1))
    return out
